# Initial kernel scaffold; baseline (speedup 1.0000x reference)
import jax
import jax.numpy as jnp
from jax.experimental import pallas as pl


def kernel(points, bbox_pred, cls_score):
    raise NotImplementedError("write your pallas kernel here")


# speedup vs baseline: 1.2873x; 1.2873x over previous
import functools

import jax
import jax.numpy as jnp
from jax.experimental import pallas as pl
from jax.experimental.pallas import tpu as pltpu

_N = 20000
_C = 18
_NMS_PRE = 1000
_PAD = 1024
_IOU_THR = 0.5
_SCORE_THR = 0.05
_BLK = 128
_NBLK = _PAD // _BLK

_INTERPRET = False


def _nms_body(boxes_r_ref, boxes_c_ref, rank_ref, score_ref, keep_out_ref,
              m_ref, p_ref, a_ref, s_ref, keep_ref):
    c = pl.program_id(0)

    @pl.when(c == 0)
    def _build_m():
        br = boxes_r_ref[...]
        bc = boxes_c_ref[...]
        volr = ((br[:, 3:4] - br[:, 0:1]) * (br[:, 4:5] - br[:, 1:2])
                * (br[:, 5:6] - br[:, 2:3]))
        volc = ((bc[3:4, :] - bc[0:1, :]) * (bc[4:5, :] - bc[1:2, :])
                * (bc[5:6, :] - bc[2:3, :]))
        for rb in range(4):
            sl = slice(rb * 256, (rb + 1) * 256)
            inter = None
            for d0 in range(3):
                il = jnp.maximum(br[sl, d0:d0 + 1], bc[d0:d0 + 1, :])
                ih = jnp.minimum(br[sl, d0 + 3:d0 + 4], bc[d0 + 3:d0 + 4, :])
                w = jnp.clip(ih - il, 0.0, None)
                inter = w if inter is None else inter * w
            union = volr[sl, :] + volc - inter
            iou = inter / jnp.maximum(union, 1e-8)
            m_ref[sl, :] = (iou > _IOU_THR).astype(jnp.bfloat16)

    rank_v = rank_ref[0]
    sc_v = score_ref[0]

    for rb in range(4):
        ri = jax.lax.broadcasted_iota(jnp.int32, (256, _PAD), 0) + rb * 256
        p_ref[rb * 256:(rb + 1) * 256, :] = (ri == rank_v).astype(jnp.bfloat16)

    for rb in range(4):
        sl = slice(rb * 256, (rb + 1) * 256)
        a_blk = jnp.dot(p_ref[sl, :], m_ref[...],
                        preferred_element_type=jnp.float32)
        a_ref[sl, :] = a_blk.astype(jnp.bfloat16)
    for rb in range(4):
        sl = slice(rb * 256, (rb + 1) * 256)
        s_blk = jax.lax.dot_general(a_ref[sl, :], p_ref[...],
                                    (((1,), (1,)), ((), ())),
                                    preferred_element_type=jnp.float32)
        s_ref[sl, :] = s_blk.astype(jnp.bfloat16)

    valid = (sc_v > _SCORE_THR).astype(jnp.bfloat16)
    vsort = jax.lax.dot_general(valid, p_ref[...], (((1,), (1,)), ((), ())),
                                preferred_element_type=jnp.float32)
    keep_ref[...] = vsort

    col = jax.lax.broadcasted_iota(jnp.int32, (1, _PAD), 1)
    lane = jax.lax.broadcasted_iota(jnp.int32, (1, _BLK), 1)

    def block_body(b, carry):
        base = pl.multiple_of(b * _BLK, _BLK)
        kb = keep_ref[:, pl.ds(base, _BLK)]
        sbb = s_ref[pl.ds(base, _BLK), pl.ds(base, _BLK)]
        for i in range(_BLK):
            srow = jax.lax.slice(sbb, (i, 0), (i + 1, _BLK))
            kti = jax.lax.slice(kb, (0, i), (1, i + 1)) > 0.5
            sup = (srow > 0.5) & kti & (lane > i)
            kb = jnp.where(sup, 0.0, kb)
        s_rows = s_ref[pl.ds(base, _BLK), :]
        supv = jax.lax.dot_general(kb.astype(jnp.bfloat16), s_rows,
                                   (((1,), (0,)), ((), ())),
                                   preferred_element_type=jnp.float32)
        kf = keep_ref[...]
        kf = jnp.where((supv > 0.5) & (col >= base + _BLK), 0.0, kf)
        keep_ref[...] = kf
        keep_ref[:, pl.ds(base, _BLK)] = kb
        return carry

    jax.lax.fori_loop(0, _NBLK, block_body, 0)

    ks = keep_ref[...].astype(jnp.bfloat16)
    korig = jax.lax.dot_general(ks, p_ref[...], (((1,), (0,)), ((), ())),
                                preferred_element_type=jnp.float32)
    keep_out_ref[0] = korig


@functools.partial(jax.jit, static_argnames=())
def _nms_pallas(boxes_r, boxes_c, rank, score):
    return pl.pallas_call(
        _nms_body,
        grid=(_C,),
        in_specs=[
            pl.BlockSpec((_PAD, 8), lambda c: (0, 0)),
            pl.BlockSpec((8, _PAD), lambda c: (0, 0)),
            pl.BlockSpec((1, 1, _PAD), lambda c: (c, 0, 0)),
            pl.BlockSpec((1, 1, _PAD), lambda c: (c, 0, 0)),
        ],
        out_specs=pl.BlockSpec((1, 1, _PAD), lambda c: (c, 0, 0)),
        out_shape=jax.ShapeDtypeStruct((_C, 1, _PAD), jnp.float32),
        scratch_shapes=[
            pltpu.VMEM((_PAD, _PAD), jnp.bfloat16),
            pltpu.VMEM((_PAD, _PAD), jnp.bfloat16),
            pltpu.VMEM((_PAD, _PAD), jnp.bfloat16),
            pltpu.VMEM((_PAD, _PAD), jnp.bfloat16),
            pltpu.VMEM((1, _PAD), jnp.float32),
        ],
        interpret=_INTERPRET,
    )(boxes_r, boxes_c, rank, score)


def kernel(points, bbox_pred, cls_score):
    scores_full = jax.nn.sigmoid(cls_score)
    max_scores = jnp.max(scores_full, axis=1)
    _, ids = jax.lax.top_k(max_scores, _NMS_PRE)
    p = points[ids]
    bp = bbox_pred[ids]
    s = scores_full[ids]
    d = jnp.exp(bp)
    lo = p - d[:, :3]
    hi = p + d[:, 3:]
    boxes = jnp.concatenate([lo, hi], axis=1)

    npad = _PAD - _NMS_PRE
    s_pad = jnp.concatenate(
        [s, jnp.full((npad, _C), -1.0, jnp.float32)], axis=0)
    boxes_pad = jnp.concatenate(
        [boxes, jnp.zeros((npad, 6), jnp.float32)], axis=0)
    order = jnp.argsort(-s_pad, axis=0)
    rank = jnp.argsort(order, axis=0)

    boxes_r = jnp.concatenate(
        [boxes_pad, jnp.zeros((_PAD, 2), jnp.float32)], axis=1)
    boxes_c = boxes_r.T
    rank_in = rank.T.reshape(_C, 1, _PAD).astype(jnp.int32)
    score_in = s_pad.T.reshape(_C, 1, _PAD)

    keep = _nms_pallas(boxes_r, boxes_c, rank_in, score_in)
    keepb = keep[:, 0, :_NMS_PRE].T > 0.5
    nms_scores = jnp.where(keepb, s, 0.0)
    return jnp.concatenate([boxes, nms_scores], axis=1)


# speedup vs baseline: 6.2275x; 4.8376x over previous
import functools

import jax
import jax.numpy as jnp
from jax.experimental import pallas as pl
from jax.experimental.pallas import tpu as pltpu

_N = 20000
_C = 18
_NMS_PRE = 1000
_PAD = 1024
_IOU_THR = 0.5
_SCORE_THR = 0.05
_BLK = 128
_NBLK = _PAD // _BLK

_INTERPRET = False


def _smat_body(boxes_r_ref, boxes_c_ref, rank_ref, s_out_ref,
               m_ref, p_ref, a_ref):
    c = pl.program_id(0)

    @pl.when(c == 0)
    def _build_m():
        br = boxes_r_ref[...]
        bc = boxes_c_ref[...]
        volr = ((br[:, 3:4] - br[:, 0:1]) * (br[:, 4:5] - br[:, 1:2])
                * (br[:, 5:6] - br[:, 2:3]))
        volc = ((bc[3:4, :] - bc[0:1, :]) * (bc[4:5, :] - bc[1:2, :])
                * (bc[5:6, :] - bc[2:3, :]))
        for rb in range(4):
            sl = slice(rb * 256, (rb + 1) * 256)
            inter = None
            for d0 in range(3):
                il = jnp.maximum(br[sl, d0:d0 + 1], bc[d0:d0 + 1, :])
                ih = jnp.minimum(br[sl, d0 + 3:d0 + 4], bc[d0 + 3:d0 + 4, :])
                w = jnp.clip(ih - il, 0.0, None)
                inter = w if inter is None else inter * w
            union = volr[sl, :] + volc - inter
            iou = inter / jnp.maximum(union, 1e-8)
            m_ref[sl, :] = (iou > _IOU_THR).astype(jnp.bfloat16)

    rank_v = rank_ref[c]

    for rb in range(4):
        ri = jax.lax.broadcasted_iota(jnp.int32, (256, _PAD), 0) + rb * 256
        p_ref[rb * 256:(rb + 1) * 256, :] = (ri == rank_v).astype(jnp.bfloat16)

    for rb in range(4):
        sl = slice(rb * 256, (rb + 1) * 256)
        a_blk = jnp.dot(p_ref[sl, :], m_ref[...],
                        preferred_element_type=jnp.float32)
        a_ref[sl, :] = a_blk.astype(jnp.bfloat16)
    for rb in range(4):
        sl = slice(rb * 256, (rb + 1) * 256)
        s_blk = jax.lax.dot_general(a_ref[sl, :], p_ref[...],
                                    (((1,), (1,)), ((), ())),
                                    preferred_element_type=jnp.float32)
        s_out_ref[0, sl, :] = s_blk.astype(jnp.bfloat16)


def _greedy_body(sall_ref, score_ref, keep_out_ref, keep_ref):
    score3 = score_ref[...]
    nvalid = jnp.sum((score3 > _SCORE_THR).astype(jnp.float32),
                     axis=2, keepdims=True)
    col3 = jax.lax.broadcasted_iota(jnp.int32, (1, _C, _PAD), 2)
    keep_ref[...] = jnp.where(col3.astype(jnp.float32) < nvalid, 1.0, 0.0)

    lane3 = jax.lax.broadcasted_iota(jnp.int32, (1, _C, _BLK), 2)

    def block_body(b, carry):
        base = pl.multiple_of(b * _BLK, _BLK)
        kb = keep_ref[:, :, pl.ds(base, _BLK)]
        acc = jnp.zeros((1, _C, _PAD), jnp.bfloat16)
        for i in range(_BLK):
            row3 = sall_ref[pl.ds(base + i, 1)]
            rowb = sall_ref[pl.ds(base + i, 1), :, pl.ds(base, _BLK)]
            kti = jax.lax.slice(kb, (0, 0, i), (1, _C, i + 1)) > 0.5
            sup = (rowb > 0.5) & kti & (lane3 > i)
            kb = jnp.where(sup, 0.0, kb)
            acc = jnp.maximum(acc, jnp.where(kti, row3, jnp.bfloat16(0)))
        kf = keep_ref[...]
        kill = (acc > 0.5) & (col3 >= base + _BLK)
        keep_ref[...] = jnp.where(kill, 0.0, kf)
        keep_ref[:, :, pl.ds(base, _BLK)] = kb
        return carry

    jax.lax.fori_loop(0, _NBLK, block_body, 0)
    keep_out_ref[...] = keep_ref[...]


@jax.jit
def _nms_pallas(boxes_r, boxes_c, rank3, score3):
    s_cmats = pl.pallas_call(
        _smat_body,
        grid=(_C,),
        in_specs=[
            pl.BlockSpec((_PAD, 8), lambda c: (0, 0)),
            pl.BlockSpec((8, _PAD), lambda c: (0, 0)),
            pl.BlockSpec((_C, 1, _PAD), lambda c: (0, 0, 0)),
        ],
        out_specs=pl.BlockSpec((1, _PAD, _PAD), lambda c: (c, 0, 0)),
        out_shape=jax.ShapeDtypeStruct((_C, _PAD, _PAD), jnp.bfloat16),
        scratch_shapes=[
            pltpu.VMEM((_PAD, _PAD), jnp.bfloat16),
            pltpu.VMEM((_PAD, _PAD), jnp.bfloat16),
            pltpu.VMEM((_PAD, _PAD), jnp.bfloat16),
        ],
        interpret=_INTERPRET,
    )(boxes_r, boxes_c, rank3)

    sall = jnp.transpose(s_cmats, (1, 0, 2))

    keep_s = pl.pallas_call(
        _greedy_body,
        in_specs=[
            pl.BlockSpec((_PAD, _C, _PAD), lambda: (0, 0, 0)),
            pl.BlockSpec((1, _C, _PAD), lambda: (0, 0, 0)),
        ],
        out_specs=pl.BlockSpec((1, _C, _PAD), lambda: (0, 0, 0)),
        out_shape=jax.ShapeDtypeStruct((1, _C, _PAD), jnp.float32),
        scratch_shapes=[pltpu.VMEM((1, _C, _PAD), jnp.float32)],
        interpret=_INTERPRET,
    )(sall, score3)
    return keep_s


def kernel(points, bbox_pred, cls_score):
    scores_full = jax.nn.sigmoid(cls_score)
    max_scores = jnp.max(scores_full, axis=1)
    _, ids = jax.lax.top_k(max_scores, _NMS_PRE)
    p = points[ids]
    bp = bbox_pred[ids]
    s = scores_full[ids]
    d = jnp.exp(bp)
    lo = p - d[:, :3]
    hi = p + d[:, 3:]
    boxes = jnp.concatenate([lo, hi], axis=1)

    npad = _PAD - _NMS_PRE
    s_pad = jnp.concatenate(
        [s, jnp.full((npad, _C), -1.0, jnp.float32)], axis=0)
    boxes_pad = jnp.concatenate(
        [boxes, jnp.zeros((npad, 6), jnp.float32)], axis=0)
    order = jnp.argsort(-s_pad, axis=0)
    rank = jnp.argsort(order, axis=0)
    s_sorted = -jnp.sort(-s_pad, axis=0)

    boxes_r = jnp.concatenate(
        [boxes_pad, jnp.zeros((_PAD, 2), jnp.float32)], axis=1)
    boxes_c = boxes_r.T
    rank3 = rank.T.reshape(_C, 1, _PAD).astype(jnp.int32)
    score3 = s_sorted.T.reshape(1, _C, _PAD)

    keep_s = _nms_pallas(boxes_r, boxes_c, rank3, score3)
    keep_orig = jnp.take_along_axis(keep_s[0], rank.T, axis=1)
    keepb = keep_orig[:, :_NMS_PRE].T > 0.5
    nms_scores = jnp.where(keepb, s, 0.0)
    return jnp.concatenate([boxes, nms_scores], axis=1)


# speedup vs baseline: 6.7207x; 1.0792x over previous
import functools

import jax
import jax.numpy as jnp
from jax.experimental import pallas as pl
from jax.experimental.pallas import tpu as pltpu

_N = 20000
_C = 18
_NMS_PRE = 1000
_PAD = 1024
_IOU_THR = 0.5
_SCORE_THR = 0.05
_BLK = 128
_NBLK = _PAD // _BLK

_INTERPRET = False


def _smat_body(boxes_r_ref, boxes_c_ref, rank_ref, s_out_ref,
               m_ref, p_ref, a_ref):
    c = pl.program_id(0)

    @pl.when(c == 0)
    def _build_m():
        br = boxes_r_ref[...]
        bc = boxes_c_ref[...]
        volr = ((br[:, 3:4] - br[:, 0:1]) * (br[:, 4:5] - br[:, 1:2])
                * (br[:, 5:6] - br[:, 2:3]))
        volc = ((bc[3:4, :] - bc[0:1, :]) * (bc[4:5, :] - bc[1:2, :])
                * (bc[5:6, :] - bc[2:3, :]))
        for rb in range(4):
            sl = slice(rb * 256, (rb + 1) * 256)
            inter = None
            for d0 in range(3):
                il = jnp.maximum(br[sl, d0:d0 + 1], bc[d0:d0 + 1, :])
                ih = jnp.minimum(br[sl, d0 + 3:d0 + 4], bc[d0 + 3:d0 + 4, :])
                w = jnp.clip(ih - il, 0.0, None)
                inter = w if inter is None else inter * w
            union = volr[sl, :] + volc - inter
            iou = inter / jnp.maximum(union, 1e-8)
            m_ref[sl, :] = (iou > _IOU_THR).astype(jnp.int8)

    rank_v = rank_ref[c]

    for rb in range(4):
        ri = jax.lax.broadcasted_iota(jnp.int32, (256, _PAD), 0) + rb * 256
        p_ref[rb * 256:(rb + 1) * 256, :] = (ri == rank_v).astype(jnp.int8)

    for rb in range(4):
        sl = slice(rb * 256, (rb + 1) * 256)
        a_blk = jnp.dot(p_ref[sl, :], m_ref[...],
                        preferred_element_type=jnp.int32)
        a_ref[sl, :] = a_blk.astype(jnp.int8)
    for rb in range(4):
        sl = slice(rb * 256, (rb + 1) * 256)
        s_blk = jax.lax.dot_general(a_ref[sl, :], p_ref[...],
                                    (((1,), (1,)), ((), ())),
                                    preferred_element_type=jnp.int32)
        s_out_ref[0, sl, :] = s_blk.astype(jnp.int8)


def _greedy_body(sall_ref, score_ref, keep_out_ref, keep_ref):
    score3 = score_ref[...]
    nvalid = jnp.sum((score3 > _SCORE_THR).astype(jnp.float32),
                     axis=2, keepdims=True)
    col3 = jax.lax.broadcasted_iota(jnp.int32, (1, _C, _PAD), 2)
    keep_ref[...] = jnp.where(col3.astype(jnp.float32) < nvalid, 1.0, 0.0)

    lane3 = jax.lax.broadcasted_iota(jnp.int32, (1, _C, _BLK), 2)

    def block_body(b, carry):
        base = pl.multiple_of(b * _BLK, _BLK)
        kb = keep_ref[:, :, pl.ds(base, _BLK)]
        acc = jnp.zeros((1, _C, _PAD), jnp.bfloat16)
        for i in range(_BLK):
            row3 = sall_ref[pl.ds(base + i, 1)].astype(jnp.bfloat16)
            rowb = sall_ref[pl.ds(base + i, 1), :,
                            pl.ds(base, _BLK)].astype(jnp.bfloat16)
            kti = jax.lax.slice(kb, (0, 0, i), (1, _C, i + 1)) > 0.5
            sup = (rowb > 0.5) & kti & (lane3 > i)
            kb = jnp.where(sup, 0.0, kb)
            acc = jnp.maximum(acc, jnp.where(kti, row3, jnp.bfloat16(0)))
        kf = keep_ref[...]
        kill = (acc > 0.5) & (col3 >= base + _BLK)
        keep_ref[...] = jnp.where(kill, 0.0, kf)
        keep_ref[:, :, pl.ds(base, _BLK)] = kb
        return carry

    jax.lax.fori_loop(0, _NBLK, block_body, 0)
    keep_out_ref[...] = keep_ref[...]


@jax.jit
def _nms_pallas(boxes_r, boxes_c, rank3, score3):
    s_cmats = pl.pallas_call(
        _smat_body,
        grid=(_C,),
        in_specs=[
            pl.BlockSpec((_PAD, 8), lambda c: (0, 0)),
            pl.BlockSpec((8, _PAD), lambda c: (0, 0)),
            pl.BlockSpec((_C, 1, _PAD), lambda c: (0, 0, 0)),
        ],
        out_specs=pl.BlockSpec((1, _PAD, _PAD), lambda c: (c, 0, 0)),
        out_shape=jax.ShapeDtypeStruct((_C, _PAD, _PAD), jnp.int8),
        scratch_shapes=[
            pltpu.VMEM((_PAD, _PAD), jnp.int8),
            pltpu.VMEM((_PAD, _PAD), jnp.int8),
            pltpu.VMEM((_PAD, _PAD), jnp.int8),
        ],
        interpret=_INTERPRET,
    )(boxes_r, boxes_c, rank3)

    sall = jnp.transpose(s_cmats, (1, 0, 2))

    keep_s = pl.pallas_call(
        _greedy_body,
        in_specs=[
            pl.BlockSpec((_PAD, _C, _PAD), lambda: (0, 0, 0)),
            pl.BlockSpec((1, _C, _PAD), lambda: (0, 0, 0)),
        ],
        out_specs=pl.BlockSpec((1, _C, _PAD), lambda: (0, 0, 0)),
        out_shape=jax.ShapeDtypeStruct((1, _C, _PAD), jnp.float32),
        scratch_shapes=[pltpu.VMEM((1, _C, _PAD), jnp.float32)],
        interpret=_INTERPRET,
    )(sall, score3)
    return keep_s


def kernel(points, bbox_pred, cls_score):
    scores_full = jax.nn.sigmoid(cls_score)
    max_scores = jnp.max(scores_full, axis=1)
    _, ids = jax.lax.top_k(max_scores, _NMS_PRE)
    p = points[ids]
    bp = bbox_pred[ids]
    s = scores_full[ids]
    d = jnp.exp(bp)
    lo = p - d[:, :3]
    hi = p + d[:, 3:]
    boxes = jnp.concatenate([lo, hi], axis=1)

    npad = _PAD - _NMS_PRE
    s_pad = jnp.concatenate(
        [s, jnp.full((npad, _C), -1.0, jnp.float32)], axis=0)
    boxes_pad = jnp.concatenate(
        [boxes, jnp.zeros((npad, 6), jnp.float32)], axis=0)
    order = jnp.argsort(-s_pad, axis=0)
    rank = jnp.argsort(order, axis=0)
    s_sorted = -jnp.sort(-s_pad, axis=0)

    boxes_r = jnp.concatenate(
        [boxes_pad, jnp.zeros((_PAD, 2), jnp.float32)], axis=1)
    boxes_c = boxes_r.T
    rank3 = rank.T.reshape(_C, 1, _PAD).astype(jnp.int32)
    score3 = s_sorted.T.reshape(1, _C, _PAD)

    keep_s = _nms_pallas(boxes_r, boxes_c, rank3, score3)
    keep_orig = jnp.take_along_axis(keep_s[0], rank.T, axis=1)
    keepb = keep_orig[:, :_NMS_PRE].T > 0.5
    nms_scores = jnp.where(keepb, s, 0.0)
    return jnp.concatenate([boxes, nms_scores], axis=1)
